# Initial kernel scaffold; baseline (speedup 1.0000x reference)
import jax
import jax.numpy as jnp
from jax.experimental import pallas as pl


def kernel(x, edge_index, W1l, W1r, b1, W2l, W2r, b2, W3, b3):
    raise NotImplementedError("write your pallas kernel here")


# speedup vs baseline: 5.3444x; 5.3444x over previous
import dataclasses
import functools

import jax
import jax.numpy as jnp
from jax import lax
from jax.experimental import pallas as pl
from jax.experimental.pallas import tpu as pltpu
from jax.experimental.pallas import tpu_sc as plsc

NC = 2
NS = 16
LANES = 16
NW = NC * NS

F32 = jnp.float32
HIGHEST = lax.Precision.HIGHEST


def _dot(a, b):
    return lax.dot_general(a, b, (((1,), (0,)), ((), ())),
                           precision=HIGHEST, preferred_element_type=F32)


def _make_sc_agg(N, E, H, with_deg):
    per_w = E // NW
    CH = 80
    n_chunks = per_w // CH
    assert per_w % CH == 0 and per_w % 8 == 0 and CH % 8 == 0
    WB = (N // NS) // 8 * 8
    REM = N - NS * WB
    ZR = 8
    assert WB % ZR == 0 and REM % ZR == 0
    ZW = 48
    assert WB % ZW == 0 and REM <= ZW

    mesh = plsc.VectorSubcoreMesh(core_axis_name="c", subcore_axis_name="s")

    out_type = [jax.ShapeDtypeStruct((NC, N, H), F32)]
    scratch = [
        pltpu.VMEM_SHARED((N, H), F32),
        pltpu.VMEM((CH,), jnp.int32),
        pltpu.VMEM((CH,), jnp.int32),
        pltpu.VMEM((CH, H), F32),
        pltpu.VMEM((ZR, H), F32),
        pltpu.VMEM((ZW, H), F32),
    ]
    if with_deg:
        out_type.append(jax.ShapeDtypeStruct((NC, NS, N), F32))
        scratch += [
            pltpu.VMEM((N,), F32),
        ]

    def body(z_hbm, src_hbm, dst_hbm, *refs):
        if with_deg:
            (out_hbm, deg_hbm, acc_sh, src_v, dst_v, rows_v, zbuf, wb_v,
             hist_v) = refs
        else:
            (out_hbm, acc_sh, src_v, dst_v, rows_v, zbuf, wb_v) = refs

        c = lax.axis_index("c")
        s = lax.axis_index("s")
        wid = s * NC + c

        @pl.loop(0, ZR)
        def _(r):
            @pl.loop(0, H // LANES)
            def _(k):
                zbuf[r, pl.ds(k * LANES, LANES)] = jnp.zeros((LANES,), F32)

        if with_deg:
            @pl.loop(0, N // LANES)
            def _(i):
                hist_v[pl.ds(i * LANES, LANES)] = jnp.zeros((LANES,), F32)

        row0 = pl.multiple_of(s * WB, 8)

        @pl.loop(0, WB // ZR)
        def _(j):
            r = pl.multiple_of(row0 + j * ZR, 8)
            pltpu.sync_copy(zbuf, acc_sh.at[pl.ds(r, ZR)])

        @pl.when(s == 0)
        def _():
            @pl.loop(0, REM // ZR)
            def _(j):
                r = pl.multiple_of(NS * WB + j * ZR, 8)
                pltpu.sync_copy(zbuf, acc_sh.at[pl.ds(r, ZR)])

        plsc.subcore_barrier()

        base0 = pl.multiple_of(wid * per_w, 8)

        @pl.loop(0, n_chunks)
        def _(i):
            b = pl.multiple_of(base0 + i * CH, 8)
            pltpu.sync_copy(src_hbm.at[pl.ds(b, CH)], src_v)
            pltpu.sync_copy(dst_hbm.at[pl.ds(b, CH)], dst_v)
            pltpu.sync_copy(z_hbm.at[src_v], rows_v)
            pltpu.sync_copy(rows_v, acc_sh.at[dst_v], add=True)
            if with_deg:
                for k in range(CH // LANES):
                    idx16 = dst_v[pl.ds(k * LANES, LANES)]
                    plsc.addupdate_scatter(hist_v, [idx16],
                                           jnp.ones((LANES,), F32))

        plsc.subcore_barrier()

        @pl.loop(0, WB // ZW)
        def _(j):
            r = pl.multiple_of(row0 + j * ZW, 8)
            pltpu.sync_copy(acc_sh.at[pl.ds(r, ZW)], wb_v)
            pltpu.sync_copy(wb_v, out_hbm.at[c, pl.ds(r, ZW)])

        @pl.when(s == 0)
        def _():
            rr = NS * WB
            pltpu.sync_copy(acc_sh.at[pl.ds(rr, REM)], wb_v.at[pl.ds(0, REM)])
            pltpu.sync_copy(wb_v.at[pl.ds(0, REM)],
                            out_hbm.at[c, pl.ds(rr, REM)])

        if with_deg:
            pltpu.sync_copy(hist_v, deg_hbm.at[c, s])

    cp = pltpu.CompilerParams()
    if "needs_layout_passes" in pltpu.CompilerParams.__dataclass_fields__:
        cp = dataclasses.replace(cp, needs_layout_passes=False)
    return pl.kernel(body, out_type=tuple(out_type), mesh=mesh,
                     scratch_types=scratch, compiler_params=cp)


_BLK = 2000


def _dense1(x, WlT, WrT, b):
    N, D = x.shape
    H = WlT.shape[1]

    def body(x_ref, wl_ref, wr_ref, b_ref, z_ref, xr_ref):
        xv = x_ref[...]
        z_ref[...] = _dot(xv, wl_ref[...])
        xr_ref[...] = _dot(xv, wr_ref[...]) + b_ref[...]

    grid = (N // _BLK,)
    return pl.pallas_call(
        body,
        grid=grid,
        in_specs=[
            pl.BlockSpec((_BLK, D), lambda i: (i, 0)),
            pl.BlockSpec((D, H), lambda i: (0, 0)),
            pl.BlockSpec((D, H), lambda i: (0, 0)),
            pl.BlockSpec((1, H), lambda i: (0, 0)),
        ],
        out_specs=[
            pl.BlockSpec((_BLK, H), lambda i: (i, 0)),
            pl.BlockSpec((_BLK, H), lambda i: (i, 0)),
        ],
        out_shape=[
            jax.ShapeDtypeStruct((N, H), F32),
            jax.ShapeDtypeStruct((N, H), F32),
        ],
    )(x, WlT, WrT, b)


def _dense_deg(degp):
    _, _, N = degp.shape

    def body(degp_ref, inv_ref):
        dv = degp_ref[...].reshape(NW, N)
        deg = lax.dot_general(dv, jnp.ones((NW, 1), F32),
                              (((0,), (0,)), ((), ())),
                              precision=HIGHEST, preferred_element_type=F32)
        inv_ref[...] = 1.0 / jnp.maximum(deg, 1.0)

    return pl.pallas_call(
        body,
        out_shape=jax.ShapeDtypeStruct((N, 1), F32),
    )(degp)


def _dense2(aggp, invd, xr1, WlT, WrT, b):
    _, N, H = aggp.shape

    def body(aggp_ref, inv_ref, xr_ref, wl_ref, wr_ref, b_ref,
             h_ref, z2_ref, hr2_ref):
        inv = inv_ref[...]
        agg = (aggp_ref[0] + aggp_ref[1]) * inv
        hv = jnp.maximum(agg + xr_ref[...], 0.0)
        h_ref[...] = hv
        z2_ref[...] = _dot(hv, wl_ref[...])
        hr2_ref[...] = _dot(hv, wr_ref[...]) + b_ref[...] + hv

    grid = (N // _BLK,)
    return pl.pallas_call(
        body,
        grid=grid,
        in_specs=[
            pl.BlockSpec((NC, _BLK, H), lambda i: (0, i, 0)),
            pl.BlockSpec((_BLK, 1), lambda i: (i, 0)),
            pl.BlockSpec((_BLK, H), lambda i: (i, 0)),
            pl.BlockSpec((H, H), lambda i: (0, 0)),
            pl.BlockSpec((H, H), lambda i: (0, 0)),
            pl.BlockSpec((1, H), lambda i: (0, 0)),
        ],
        out_specs=[
            pl.BlockSpec((_BLK, H), lambda i: (i, 0)),
            pl.BlockSpec((_BLK, H), lambda i: (i, 0)),
            pl.BlockSpec((_BLK, H), lambda i: (i, 0)),
        ],
        out_shape=[
            jax.ShapeDtypeStruct((N, H), F32),
            jax.ShapeDtypeStruct((N, H), F32),
            jax.ShapeDtypeStruct((N, H), F32),
        ],
    )(aggp, invd, xr1, WlT, WrT, b)


def _dense3(aggp, invd, hr2, W3Tp, b3p, C):
    _, N, H = aggp.shape

    def body(aggp_ref, inv_ref, hr_ref, w3_ref, b3_ref, out_ref):
        inv = inv_ref[...]
        h_out = (aggp_ref[0] + aggp_ref[1]) * inv + hr_ref[...]
        logits = _dot(h_out, w3_ref[...]) + b3_ref[...]
        mask = lax.broadcasted_iota(jnp.int32, logits.shape, 1) < C
        masked = jnp.where(mask, logits, -1e30)
        m = jnp.max(masked, axis=1, keepdims=True)
        ex = jnp.where(mask, jnp.exp(logits - m), 0.0)
        lse = jnp.log(jnp.sum(ex, axis=1, keepdims=True)) + m
        res = logits - lse
        out_ref[...] = res[:, :C]

    grid = (N // _BLK,)
    return pl.pallas_call(
        body,
        grid=grid,
        in_specs=[
            pl.BlockSpec((NC, _BLK, H), lambda i: (0, i, 0)),
            pl.BlockSpec((_BLK, 1), lambda i: (i, 0)),
            pl.BlockSpec((_BLK, H), lambda i: (i, 0)),
            pl.BlockSpec((H, H), lambda i: (0, 0)),
            pl.BlockSpec((1, H), lambda i: (0, 0)),
        ],
        out_specs=pl.BlockSpec((_BLK, C), lambda i: (i, 0)),
        out_shape=jax.ShapeDtypeStruct((N, C), F32),
    )(aggp, invd, hr2, W3Tp, b3p)


def kernel(x, edge_index, W1l, W1r, b1, W2l, W2r, b2, W3, b3):
    N, D = x.shape
    E = edge_index.shape[1]
    H = W1l.shape[0]
    C = W3.shape[0]

    src = edge_index[0]
    dst = edge_index[1]

    sc_agg_deg = _make_sc_agg(N, E, H, with_deg=True)
    sc_agg = _make_sc_agg(N, E, H, with_deg=False)

    z1, xr1 = _dense1(x, W1l.T, W1r.T, b1.reshape(1, H))
    aggp1, degp = sc_agg_deg(z1, src, dst)
    invd = _dense_deg(degp)
    h, z2, hr2 = _dense2(aggp1, invd, xr1, W2l.T, W2r.T, b2.reshape(1, H))

    (aggp2,) = sc_agg(z2, src, dst)
    W3Tp = jnp.zeros((H, H), F32).at[:, :C].set(W3.T)
    b3p = jnp.zeros((1, H), F32).at[0, :C].set(b3)
    out = _dense3(aggp2, invd, hr2, W3Tp, b3p, C)

    return (out, edge_index)


# speedup vs baseline: 9.3886x; 1.7567x over previous
import dataclasses
import functools

import jax
import jax.numpy as jnp
from jax import lax
from jax.experimental import pallas as pl
from jax.experimental.pallas import tpu as pltpu
from jax.experimental.pallas import tpu_sc as plsc

NC = 2
NS = 16
LANES = 16
NW = NC * NS

F32 = jnp.float32
HIGHEST = lax.Precision.HIGHEST


def _dot(a, b):
    return lax.dot_general(a, b, (((1,), (0,)), ((), ())),
                           precision=HIGHEST, preferred_element_type=F32)


def _make_sc_agg(N, E, H, with_deg):
    per_w = E // NW
    CH = 80
    n_chunks = per_w // CH
    assert per_w % CH == 0 and per_w % 8 == 0 and CH % 8 == 0
    WB = (N // NS) // 8 * 8
    REM = N - NS * WB
    ZR = 8
    assert WB % ZR == 0 and REM % ZR == 0
    ZW = 48
    assert WB % ZW == 0 and REM <= ZW

    mesh = plsc.VectorSubcoreMesh(core_axis_name="c", subcore_axis_name="s")

    QUADS = n_chunks // 4
    REMC = n_chunks - QUADS * 4
    assert REMC == 1

    out_type = [jax.ShapeDtypeStruct((NC, N, H), F32)]
    scratch = [
        pltpu.VMEM_SHARED((N, H), F32),
        [pltpu.VMEM((CH,), jnp.int32) for _ in range(4)],
        [pltpu.VMEM((CH,), jnp.int32) for _ in range(4)],
        pltpu.VMEM((CH, H), F32),
        pltpu.VMEM((CH, H), F32),
        pltpu.VMEM((ZR, H), F32),
        pltpu.VMEM((ZW, H), F32),
        pltpu.SemaphoreType.DMA,
        pltpu.SemaphoreType.DMA,
        pltpu.SemaphoreType.DMA,
        pltpu.SemaphoreType.DMA,
        pltpu.SemaphoreType.DMA,
    ]
    if with_deg:
        out_type.append(jax.ShapeDtypeStruct((NC, NS, N), F32))
        scratch += [
            pltpu.VMEM((N,), F32),
        ]

    def body(z_hbm, src_hbm, dst_hbm, *refs):
        if with_deg:
            (out_hbm, deg_hbm, acc_sh, src_v, dst_v, rows_a, rows_b, zbuf,
             wb_v, si, sg_a, sg_b, ss_a, ss_b, hist_v) = refs
        else:
            (out_hbm, acc_sh, src_v, dst_v, rows_a, rows_b, zbuf,
             wb_v, si, sg_a, sg_b, ss_a, ss_b) = refs

        c = lax.axis_index("c")
        s = lax.axis_index("s")
        wid = s * NC + c

        @pl.loop(0, ZR)
        def _(r):
            @pl.loop(0, H // LANES)
            def _(k):
                zbuf[r, pl.ds(k * LANES, LANES)] = jnp.zeros((LANES,), F32)

        if with_deg:
            @pl.loop(0, N // LANES)
            def _(i):
                hist_v[pl.ds(i * LANES, LANES)] = jnp.zeros((LANES,), F32)

        row0 = pl.multiple_of(s * WB, 8)

        @pl.loop(0, WB // ZR)
        def _(j):
            r = pl.multiple_of(row0 + j * ZR, 8)
            pltpu.sync_copy(zbuf, acc_sh.at[pl.ds(r, ZR)])

        @pl.when(s == 0)
        def _():
            @pl.loop(0, REM // ZR)
            def _(j):
                r = pl.multiple_of(NS * WB + j * ZR, 8)
                pltpu.sync_copy(zbuf, acc_sh.at[pl.ds(r, ZR)])

        plsc.subcore_barrier()

        base0 = pl.multiple_of(wid * per_w, 8)

        def hist_update(dv):
            if with_deg:
                for k in range(CH // LANES):
                    idx16 = dv[pl.ds(k * LANES, LANES)]
                    plsc.addupdate_scatter(hist_v, [idx16],
                                           jnp.ones((LANES,), F32))

        def load_idx(j, b):
            return (pltpu.async_copy(src_hbm.at[pl.ds(b, CH)], src_v[j], si),
                    pltpu.async_copy(dst_hbm.at[pl.ds(b, CH)], dst_v[j], si))

        @pl.loop(0, QUADS)
        def _(k):
            @pl.when(k > 0)
            def _():
                pltpu.make_async_copy(rows_b, acc_sh.at[dst_v[3]],
                                      ss_b).wait()

            c0 = pl.multiple_of(base0 + k * 4 * CH, 8)
            ds = [load_idx(j, pl.multiple_of(c0 + j * CH, 8))
                  for j in range(4)]
            for dsrc, ddst in ds:
                dsrc.wait()
                ddst.wait()

            g0 = pltpu.async_copy(z_hbm.at[src_v[0]], rows_a, sg_a)
            g1 = pltpu.async_copy(z_hbm.at[src_v[1]], rows_b, sg_b)
            g0.wait()
            s0 = pltpu.async_copy(rows_a, acc_sh.at[dst_v[0]], ss_a, add=True)
            hist_update(dst_v[0])
            s0.wait()
            g2 = pltpu.async_copy(z_hbm.at[src_v[2]], rows_a, sg_a)
            g1.wait()
            s1 = pltpu.async_copy(rows_b, acc_sh.at[dst_v[1]], ss_b, add=True)
            hist_update(dst_v[1])
            s1.wait()
            g3 = pltpu.async_copy(z_hbm.at[src_v[3]], rows_b, sg_b)
            g2.wait()
            s2 = pltpu.async_copy(rows_a, acc_sh.at[dst_v[2]], ss_a, add=True)
            hist_update(dst_v[2])
            g3.wait()
            pltpu.async_copy(rows_b, acc_sh.at[dst_v[3]], ss_b, add=True)
            hist_update(dst_v[3])
            s2.wait()

        pltpu.make_async_copy(rows_b, acc_sh.at[dst_v[3]], ss_b).wait()
        br = pl.multiple_of(base0 + QUADS * 4 * CH, 8)
        dsrc, ddst = load_idx(0, br)
        dsrc.wait()
        ddst.wait()
        pltpu.async_copy(z_hbm.at[src_v[0]], rows_a, sg_a).wait()
        sr = pltpu.async_copy(rows_a, acc_sh.at[dst_v[0]], ss_a, add=True)
        hist_update(dst_v[0])
        sr.wait()

        plsc.subcore_barrier()

        @pl.loop(0, WB // ZW)
        def _(j):
            r = pl.multiple_of(row0 + j * ZW, 8)
            pltpu.sync_copy(acc_sh.at[pl.ds(r, ZW)], wb_v)
            pltpu.sync_copy(wb_v, out_hbm.at[c, pl.ds(r, ZW)])

        @pl.when(s == 0)
        def _():
            rr = NS * WB
            pltpu.sync_copy(acc_sh.at[pl.ds(rr, REM)], wb_v.at[pl.ds(0, REM)])
            pltpu.sync_copy(wb_v.at[pl.ds(0, REM)],
                            out_hbm.at[c, pl.ds(rr, REM)])

        if with_deg:
            pltpu.sync_copy(hist_v, deg_hbm.at[c, s])

    cp = pltpu.CompilerParams()
    if "needs_layout_passes" in pltpu.CompilerParams.__dataclass_fields__:
        cp = dataclasses.replace(cp, needs_layout_passes=False)
    return pl.kernel(body, out_type=tuple(out_type), mesh=mesh,
                     scratch_types=scratch, compiler_params=cp)


_BLK = 2000


def _dense1(x, WlT, WrT, b):
    N, D = x.shape
    H = WlT.shape[1]

    def body(x_ref, wl_ref, wr_ref, b_ref, z_ref, xr_ref):
        xv = x_ref[...]
        z_ref[...] = _dot(xv, wl_ref[...])
        xr_ref[...] = _dot(xv, wr_ref[...]) + b_ref[...]

    grid = (N // _BLK,)
    return pl.pallas_call(
        body,
        grid=grid,
        in_specs=[
            pl.BlockSpec((_BLK, D), lambda i: (i, 0)),
            pl.BlockSpec((D, H), lambda i: (0, 0)),
            pl.BlockSpec((D, H), lambda i: (0, 0)),
            pl.BlockSpec((1, H), lambda i: (0, 0)),
        ],
        out_specs=[
            pl.BlockSpec((_BLK, H), lambda i: (i, 0)),
            pl.BlockSpec((_BLK, H), lambda i: (i, 0)),
        ],
        out_shape=[
            jax.ShapeDtypeStruct((N, H), F32),
            jax.ShapeDtypeStruct((N, H), F32),
        ],
    )(x, WlT, WrT, b)


def _dense_deg(degp):
    _, _, N = degp.shape

    def body(degp_ref, inv_ref):
        dv = degp_ref[...].reshape(NW, N)
        deg = lax.dot_general(dv, jnp.ones((NW, 1), F32),
                              (((0,), (0,)), ((), ())),
                              precision=HIGHEST, preferred_element_type=F32)
        inv_ref[...] = 1.0 / jnp.maximum(deg, 1.0)

    return pl.pallas_call(
        body,
        out_shape=jax.ShapeDtypeStruct((N, 1), F32),
    )(degp)


def _dense2(aggp, invd, xr1, WlT, WrT, b):
    _, N, H = aggp.shape

    def body(aggp_ref, inv_ref, xr_ref, wl_ref, wr_ref, b_ref,
             h_ref, z2_ref, hr2_ref):
        inv = inv_ref[...]
        agg = (aggp_ref[0] + aggp_ref[1]) * inv
        hv = jnp.maximum(agg + xr_ref[...], 0.0)
        h_ref[...] = hv
        z2_ref[...] = _dot(hv, wl_ref[...])
        hr2_ref[...] = _dot(hv, wr_ref[...]) + b_ref[...] + hv

    grid = (N // _BLK,)
    return pl.pallas_call(
        body,
        grid=grid,
        in_specs=[
            pl.BlockSpec((NC, _BLK, H), lambda i: (0, i, 0)),
            pl.BlockSpec((_BLK, 1), lambda i: (i, 0)),
            pl.BlockSpec((_BLK, H), lambda i: (i, 0)),
            pl.BlockSpec((H, H), lambda i: (0, 0)),
            pl.BlockSpec((H, H), lambda i: (0, 0)),
            pl.BlockSpec((1, H), lambda i: (0, 0)),
        ],
        out_specs=[
            pl.BlockSpec((_BLK, H), lambda i: (i, 0)),
            pl.BlockSpec((_BLK, H), lambda i: (i, 0)),
            pl.BlockSpec((_BLK, H), lambda i: (i, 0)),
        ],
        out_shape=[
            jax.ShapeDtypeStruct((N, H), F32),
            jax.ShapeDtypeStruct((N, H), F32),
            jax.ShapeDtypeStruct((N, H), F32),
        ],
    )(aggp, invd, xr1, WlT, WrT, b)


def _dense3(aggp, invd, hr2, W3Tp, b3p, C):
    _, N, H = aggp.shape

    def body(aggp_ref, inv_ref, hr_ref, w3_ref, b3_ref, out_ref):
        inv = inv_ref[...]
        h_out = (aggp_ref[0] + aggp_ref[1]) * inv + hr_ref[...]
        logits = _dot(h_out, w3_ref[...]) + b3_ref[...]
        mask = lax.broadcasted_iota(jnp.int32, logits.shape, 1) < C
        masked = jnp.where(mask, logits, -1e30)
        m = jnp.max(masked, axis=1, keepdims=True)
        ex = jnp.where(mask, jnp.exp(logits - m), 0.0)
        lse = jnp.log(jnp.sum(ex, axis=1, keepdims=True)) + m
        res = logits - lse
        out_ref[...] = res[:, :C]

    grid = (N // _BLK,)
    return pl.pallas_call(
        body,
        grid=grid,
        in_specs=[
            pl.BlockSpec((NC, _BLK, H), lambda i: (0, i, 0)),
            pl.BlockSpec((_BLK, 1), lambda i: (i, 0)),
            pl.BlockSpec((_BLK, H), lambda i: (i, 0)),
            pl.BlockSpec((H, H), lambda i: (0, 0)),
            pl.BlockSpec((1, H), lambda i: (0, 0)),
        ],
        out_specs=pl.BlockSpec((_BLK, C), lambda i: (i, 0)),
        out_shape=jax.ShapeDtypeStruct((N, C), F32),
    )(aggp, invd, hr2, W3Tp, b3p)


def kernel(x, edge_index, W1l, W1r, b1, W2l, W2r, b2, W3, b3):
    N, D = x.shape
    E = edge_index.shape[1]
    H = W1l.shape[0]
    C = W3.shape[0]

    src = edge_index[0]
    dst = edge_index[1]

    sc_agg_deg = _make_sc_agg(N, E, H, with_deg=True)
    sc_agg = _make_sc_agg(N, E, H, with_deg=False)

    z1, xr1 = _dense1(x, W1l.T, W1r.T, b1.reshape(1, H))
    aggp1, degp = sc_agg_deg(z1, src, dst)
    invd = _dense_deg(degp)
    h, z2, hr2 = _dense2(aggp1, invd, xr1, W2l.T, W2r.T, b2.reshape(1, H))

    (aggp2,) = sc_agg(z2, src, dst)
    W3Tp = jnp.zeros((H, H), F32).at[:, :C].set(W3.T)
    b3p = jnp.zeros((1, H), F32).at[0, :C].set(b3)
    out = _dense3(aggp2, invd, hr2, W3Tp, b3p, C)

    return (out, edge_index)
